# baseline (speedup 1.0000x reference)
import functools

import jax
import jax.numpy as jnp
from jax import lax
from jax.experimental import pallas as pl
from jax.experimental.pallas import tpu as pltpu
from jax.experimental.pallas import tpu_sc as plsc

_INFO = plsc.get_sparse_core_info()
_NC = _INFO.num_cores
_NS = _INFO.num_subcores
_NL = _INFO.num_lanes
_NW = _NC * _NS
_IDX_PER_DMA = 128


@functools.lru_cache(maxsize=None)
def _make_gather(n: int, vocab: int):
    assert n % (_NW * _IDX_PER_DMA) == 0
    chunk = n // _NW
    ndma = chunk // _IDX_PER_DMA
    nvec = chunk // _NL

    mesh = plsc.VectorSubcoreMesh(core_axis_name="c", subcore_axis_name="s")

    @functools.partial(
        pl.kernel,
        mesh=mesh,
        out_type=jax.ShapeDtypeStruct((n,), jnp.float32),
        scratch_types=[
            pltpu.VMEM((chunk,), jnp.int32),
            pltpu.VMEM((ndma, _IDX_PER_DMA), jnp.int32),
            pltpu.VMEM((ndma, _IDX_PER_DMA), jnp.float32),
            pltpu.SemaphoreType.DMA,
        ],
    )
    def gather(pred_hbm, tgt_hbm, out_hbm, tgt_v, idx_v, val_v, sem):
        wid = lax.axis_index("s") * _NC + lax.axis_index("c")
        base = wid * chunk
        pltpu.sync_copy(tgt_hbm.at[pl.ds(base, chunk)], tgt_v)
        for j in range(nvec):
            rows = base + j * _NL + lax.broadcasted_iota(jnp.int32, (_NL,), 0)
            idx_v[j * _NL // _IDX_PER_DMA,
                  pl.ds((j * _NL) % _IDX_PER_DMA, _NL)] = (
                tgt_v[pl.ds(j * _NL, _NL)] + rows * vocab)
        copies = [
            pltpu.async_copy(pred_hbm.at[idx_v.at[k]], val_v.at[k], sem)
            for k in range(ndma)
        ]
        for cp in copies:
            cp.wait()
        for k in range(ndma):
            pltpu.sync_copy(
                val_v.at[k],
                out_hbm.at[pl.ds(base + k * _IDX_PER_DMA, _IDX_PER_DMA)])

    return gather


def kernel(prediction, tgt):
    b, t, vocab = prediction.shape
    if t < tgt.shape[1]:
        zeros = jnp.zeros((b, tgt.shape[1] - t, vocab), dtype=prediction.dtype)
        prediction = jnp.concatenate((prediction, zeros), axis=1)
    pred_flat = prediction.reshape(-1)
    tgt_flat = tgt.reshape(-1).astype(jnp.int32)
    out = _make_gather(tgt_flat.shape[0], vocab)(pred_flat, tgt_flat)
    return out.reshape(tgt.shape)


# speedup vs baseline: 5.2274x; 5.2274x over previous
import functools

import jax
import jax.numpy as jnp
from jax import lax
from jax.experimental import pallas as pl
from jax.experimental.pallas import tpu as pltpu
from jax.experimental.pallas import tpu_sc as plsc

_INFO = plsc.get_sparse_core_info()
_NC = _INFO.num_cores
_NS = _INFO.num_subcores
_NL = _INFO.num_lanes
_NW = _NC * _NS
_IDX_PER_DMA = 128


@functools.lru_cache(maxsize=None)
def _make_gather(n: int, seq: int, vocab: int):
    assert n % (_NW * _IDX_PER_DMA) == 0 and seq % 8 == 0 and vocab % 128 == 0
    chunk = n // _NW
    ndma = chunk // _IDX_PER_DMA
    nvec = chunk // _NL

    mesh = plsc.VectorSubcoreMesh(core_axis_name="c", subcore_axis_name="s")

    @functools.partial(
        pl.kernel,
        mesh=mesh,
        out_type=jax.ShapeDtypeStruct((n,), jnp.float32),
        scratch_types=[
            pltpu.VMEM((chunk,), jnp.int32),
            pltpu.VMEM((ndma, _IDX_PER_DMA), jnp.int32),
            pltpu.VMEM((ndma, _IDX_PER_DMA), jnp.float32),
            pltpu.SemaphoreType.DMA,
        ],
    )
    def gather(pred_hbm, tgt_hbm, out_hbm, tgt_v, idx_v, val_v, sem):
        wid = lax.axis_index("s") * _NC + lax.axis_index("c")
        base = wid * chunk
        pltpu.sync_copy(tgt_hbm.at[pl.ds(base, chunk)], tgt_v)
        for j in range(nvec):
            pos = base + j * _NL + lax.broadcasted_iota(jnp.int32, (_NL,), 0)
            v = tgt_v[pl.ds(j * _NL, _NL)]
            idx_v[j * _NL // _IDX_PER_DMA,
                  pl.ds((j * _NL) % _IDX_PER_DMA, _NL)] = (
                ((pos >> 3) * (vocab // 128) + (v >> 7)) * 1024
                + (pos & 7) * 128 + (v & 127))
        copies = [
            pltpu.async_copy(pred_hbm.at[idx_v.at[k]], val_v.at[k], sem)
            for k in range(ndma)
        ]
        for cp in copies:
            cp.wait()
        for k in range(ndma):
            pltpu.sync_copy(
                val_v.at[k],
                out_hbm.at[pl.ds(base + k * _IDX_PER_DMA, _IDX_PER_DMA)])

    return gather


def kernel(prediction, tgt):
    b, t, vocab = prediction.shape
    if t < tgt.shape[1]:
        zeros = jnp.zeros((b, tgt.shape[1] - t, vocab), dtype=prediction.dtype)
        prediction = jnp.concatenate((prediction, zeros), axis=1)
    t = prediction.shape[1]
    pred_lin = (prediction
                .reshape(b, t // 8, 8, vocab // 128, 128)
                .transpose(0, 1, 3, 2, 4)
                .reshape(-1))
    tgt_flat = tgt.reshape(-1).astype(jnp.int32)
    out = _make_gather(tgt_flat.shape[0], t, vocab)(pred_lin, tgt_flat)
    return out.reshape(tgt.shape)


# speedup vs baseline: 5.5866x; 1.0687x over previous
import functools

import jax
import jax.numpy as jnp
from jax import lax
from jax.experimental import pallas as pl
from jax.experimental.pallas import tpu as pltpu
from jax.experimental.pallas import tpu_sc as plsc

_INFO = plsc.get_sparse_core_info()
_NC = _INFO.num_cores
_NS = _INFO.num_subcores
_NL = _INFO.num_lanes
_NW = _NC * _NS
_IDX_PER_DMA = 128


@functools.lru_cache(maxsize=None)
def _make_gather(bsz: int, seq: int, vocab: int):
    n = bsz * seq
    assert n % (_NW * _IDX_PER_DMA) == 0 and seq % 8 == 0 and vocab % 128 == 0
    chunk = n // _NW
    ndma = chunk // _IDX_PER_DMA
    nvec = chunk // _NL
    epb = seq // chunk
    assert seq % chunk == 0 and epb & (epb - 1) == 0
    eshift = epb.bit_length() - 1

    mesh = plsc.VectorSubcoreMesh(core_axis_name="c", subcore_axis_name="s")

    @functools.partial(
        pl.kernel,
        mesh=mesh,
        out_type=jax.ShapeDtypeStruct((bsz, seq), jnp.float32),
        scratch_types=[
            pltpu.VMEM((chunk,), jnp.int32),
            pltpu.VMEM((ndma, _IDX_PER_DMA), jnp.int32),
            pltpu.VMEM((ndma, _IDX_PER_DMA), jnp.float32),
            pltpu.SemaphoreType.DMA,
        ],
    )
    def gather(pred_hbm, tgt_hbm, out_hbm, tgt_v, idx_v, val_v, sem):
        wid = lax.axis_index("s") * _NC + lax.axis_index("c")
        bid = wid >> eshift
        t0 = (wid & (epb - 1)) * chunk
        base = wid * chunk
        pltpu.sync_copy(tgt_hbm.at[bid, pl.ds(t0, chunk)], tgt_v)
        for j in range(nvec):
            pos = base + j * _NL + lax.broadcasted_iota(jnp.int32, (_NL,), 0)
            v = tgt_v[pl.ds(j * _NL, _NL)]
            idx_v[j * _NL // _IDX_PER_DMA,
                  pl.ds((j * _NL) % _IDX_PER_DMA, _NL)] = (
                ((pos >> 3) * (vocab // 128) + (v >> 7)) * 1024
                + (pos & 7) * 128 + (v & 127))
        copies = [
            pltpu.async_copy(pred_hbm.at[idx_v.at[k]], val_v.at[k], sem)
            for k in range(ndma)
        ]
        for cp in copies:
            cp.wait()
        for k in range(ndma):
            pltpu.sync_copy(
                val_v.at[k],
                out_hbm.at[bid, pl.ds(t0 + k * _IDX_PER_DMA, _IDX_PER_DMA)])

    return gather


def kernel(prediction, tgt):
    b, t, vocab = prediction.shape
    if t < tgt.shape[1]:
        zeros = jnp.zeros((b, tgt.shape[1] - t, vocab), dtype=prediction.dtype)
        prediction = jnp.concatenate((prediction, zeros), axis=1)
        t = tgt.shape[1]
    pred_lin = (prediction
                .reshape(b, t // 8, 8, vocab // 128, 128)
                .transpose(0, 1, 3, 2, 4)
                .reshape(-1))
    return _make_gather(b, t, vocab)(pred_lin, tgt.astype(jnp.int32))


# speedup vs baseline: 6.0269x; 1.0788x over previous
import functools

import jax
import jax.numpy as jnp
from jax import lax
from jax.experimental import pallas as pl
from jax.experimental.pallas import tpu as pltpu
from jax.experimental.pallas import tpu_sc as plsc

_INFO = plsc.get_sparse_core_info()
_NS = _INFO.num_subcores
_NL = _INFO.num_lanes
_IDX_PER_DMA = 128


@functools.lru_cache(maxsize=None)
def _make_gather(bsz: int, seq: int, vocab: int):
    n = bsz * seq
    nw = _NS
    assert n % (nw * _IDX_PER_DMA) == 0 and seq % 8 == 0 and vocab % 128 == 0
    chunk = n // nw
    ndma = chunk // _IDX_PER_DMA
    nvec = chunk // _NL
    epb = seq // chunk
    assert seq % chunk == 0 and epb & (epb - 1) == 0
    eshift = epb.bit_length() - 1

    mesh = plsc.VectorSubcoreMesh(core_axis_name="c", subcore_axis_name="s",
                                  num_cores=1)

    @functools.partial(
        pl.kernel,
        mesh=mesh,
        out_type=jax.ShapeDtypeStruct((bsz, seq), jnp.float32),
        scratch_types=[
            pltpu.VMEM((chunk,), jnp.int32),
            pltpu.VMEM((ndma, _IDX_PER_DMA), jnp.int32),
            pltpu.VMEM((chunk,), jnp.float32),
            pltpu.SemaphoreType.DMA,
            pltpu.SemaphoreType.DMA,
        ],
    )
    def gather(pred_hbm, tgt_hbm, out_hbm, tgt_v, idx_v, val_v, gsem, ssem):
        wid = lax.axis_index("s")
        bid = wid >> eshift
        t0 = (wid & (epb - 1)) * chunk
        base = wid * chunk
        vpd = _IDX_PER_DMA // _NL
        stages = [
            pltpu.async_copy(
                tgt_hbm.at[bid, pl.ds(t0 + k * _IDX_PER_DMA, _IDX_PER_DMA)],
                tgt_v.at[pl.ds(k * _IDX_PER_DMA, _IDX_PER_DMA)], ssem)
            for k in range(ndma)
        ]
        gathers = []
        for k in range(ndma):
            stages[k].wait()
            for jj in range(vpd):
                j = k * vpd + jj
                pos = (base + j * _NL
                       + lax.broadcasted_iota(jnp.int32, (_NL,), 0))
                v = tgt_v[pl.ds(j * _NL, _NL)]
                idx_v[k, pl.ds(jj * _NL, _NL)] = (
                    ((pos >> 3) * (vocab // 128) + (v >> 7)) * 1024
                    + (pos & 7) * 128 + (v & 127))
            gathers.append(pltpu.async_copy(
                pred_hbm.at[idx_v.at[k]],
                val_v.at[pl.ds(k * _IDX_PER_DMA, _IDX_PER_DMA)], gsem))
        writes = []
        for k in range(ndma):
            gathers[k].wait()
            writes.append(pltpu.async_copy(
                val_v.at[pl.ds(k * _IDX_PER_DMA, _IDX_PER_DMA)],
                out_hbm.at[bid, pl.ds(t0 + k * _IDX_PER_DMA, _IDX_PER_DMA)],
                ssem))
        for cp in writes:
            cp.wait()

    return gather


def kernel(prediction, tgt):
    b, t, vocab = prediction.shape
    if t < tgt.shape[1]:
        zeros = jnp.zeros((b, tgt.shape[1] - t, vocab), dtype=prediction.dtype)
        prediction = jnp.concatenate((prediction, zeros), axis=1)
        t = tgt.shape[1]
    pred_lin = (prediction
                .reshape(b, t // 8, 8, vocab // 128, 128)
                .transpose(0, 1, 3, 2, 4)
                .reshape(-1))
    return _make_gather(b, t, vocab)(pred_lin, tgt.astype(jnp.int32))
